# baseline (speedup 1.0000x reference)
import jax
import jax.numpy as jnp
import numpy as np
from jax.experimental import pallas as pl

_PATCH = 16
_NPS = 32

_KEEP_BITS_HEX = [
    0x36eadc9b, 0x6db41695, 0xab1ba7bb, 0x6ee7587b,
    0x16d82f89, 0x71d063b6, 0x69ab3a93, 0x7339a0b9,
    0x8e82277b, 0x14fdcc8a, 0x1e6a6284, 0xdf0e4208,
    0x243af85f, 0x1d7ccc04, 0xe52d395f, 0xc619ad56,
    0x2fd3344b, 0x450e09d3, 0x3bfa5e0d, 0x123fe3f5,
    0xf750ca43, 0xe8299b1c, 0x24baa733, 0x1d15fc6f,
    0x410732a4, 0xa48fd812, 0xe4ee24d4, 0xc6fbd063,
    0x33412a1d, 0x10e63c49, 0x7ed280a9, 0xf411ae0e,
]

_KEEP_PATCH = np.array(
    [[(b >> c) & 1 for c in range(_NPS)] for b in _KEEP_BITS_HEX],
    dtype=np.float32,
)
_KEEP_FULL = np.kron(_KEEP_PATCH, np.ones((_PATCH, _PATCH), np.float32))


def _body(mask_ref, img_ref, out_ref):
    out_ref[...] = img_ref[...]


def kernel(img):
    c, h, w = img.shape
    cb = 12
    mask = jnp.asarray(_KEEP_FULL)
    return pl.pallas_call(
        _body,
        grid=(c // cb,),
        in_specs=[
            pl.BlockSpec((h, w), lambda i: (0, 0)),
            pl.BlockSpec((cb, h, w), lambda i: (i, 0, 0)),
        ],
        out_specs=pl.BlockSpec((cb, h, w), lambda i: (i, 0, 0)),
        out_shape=jax.ShapeDtypeStruct((c, h, w), img.dtype),
    )(mask, img)


# speedup vs baseline: 1.0027x; 1.0027x over previous
import jax
import jax.numpy as jnp
import numpy as np
from jax.experimental import pallas as pl

_PATCH = 16
_NPS = 32

_KEEP_BITS_HEX = [
    0x36eadc9b, 0x6db41695, 0xab1ba7bb, 0x6ee7587b,
    0x16d82f89, 0x71d063b6, 0x69ab3a93, 0x7339a0b9,
    0x8e82277b, 0x14fdcc8a, 0x1e6a6284, 0xdf0e4208,
    0x243af85f, 0x1d7ccc04, 0xe52d395f, 0xc619ad56,
    0x2fd3344b, 0x450e09d3, 0x3bfa5e0d, 0x123fe3f5,
    0xf750ca43, 0xe8299b1c, 0x24baa733, 0x1d15fc6f,
    0x410732a4, 0xa48fd812, 0xe4ee24d4, 0xc6fbd063,
    0x33412a1d, 0x10e63c49, 0x7ed280a9, 0xf411ae0e,
]

_KEEP_PATCH = np.array(
    [[(b >> c) & 1 for c in range(_NPS)] for b in _KEEP_BITS_HEX],
    dtype=np.float32,
)
_KEEP_FULL = np.kron(_KEEP_PATCH, np.ones((_PATCH, _PATCH), np.float32))


def _body(mask_ref, img_ref, out_ref):
    out_ref[...] = img_ref[...] * mask_ref[...][None, :, :]


def kernel(img):
    c, h, w = img.shape
    cb = 12
    mask = jnp.asarray(_KEEP_FULL)
    return pl.pallas_call(
        _body,
        grid=(c // cb,),
        in_specs=[
            pl.BlockSpec((h, w), lambda i: (0, 0)),
            pl.BlockSpec((cb, h, w), lambda i: (i, 0, 0)),
        ],
        out_specs=pl.BlockSpec((cb, h, w), lambda i: (i, 0, 0)),
        out_shape=jax.ShapeDtypeStruct((c, h, w), img.dtype),
    )(mask, img)
